# baseline (speedup 1.0000x reference)
import functools

import jax
import jax.numpy as jnp
from jax import lax
from jax.experimental import pallas as pl
from jax.experimental.pallas import tpu as pltpu
from jax.experimental.pallas import tpu_sc as plsc

_N, _K, _C, _OC, _KS = 10000, 32, 128, 128, 32
_NW = 32
_P = 320
_NPAD = _NW * _P
_G = 4
_NG = _P // _G
_R = _G * _K


def _sc_conv(inputs, nnidx, filt, cnt16, sw):
  mesh = plsc.VectorSubcoreMesh(core_axis_name="c", subcore_axis_name="s")

  @functools.partial(
      pl.kernel,
      mesh=mesh,
      out_type=jax.ShapeDtypeStruct((_NPAD, _C), jnp.float32),
      compiler_params=pltpu.CompilerParams(needs_layout_passes=False),
      scratch_types=[
          pltpu.VMEM((_P * _K,), jnp.int32),
          pltpu.VMEM((_P * _K,), jnp.int32),
          pltpu.VMEM((_NG, 16), jnp.int32),
          pltpu.VMEM((_KS * _C,), jnp.float32),
          pltpu.VMEM((4, _R, _C), jnp.float32),
          pltpu.VMEM((4, _G, _C), jnp.float32),
          pltpu.SemaphoreType.DMA,
          pltpu.SemaphoreType.DMA,
          pltpu.SemaphoreType.DMA,
          pltpu.SemaphoreType.DMA,
          pltpu.SemaphoreType.DMA,
          pltpu.SemaphoreType.DMA,
          pltpu.SemaphoreType.DMA,
          pltpu.SemaphoreType.DMA,
      ],
  )
  def body(inp, nni, fli, c16, swr, out, idx_v, fid_v, cnt_v, swl, nb, ob,
           sem_n0, sem_n1, sem_n2, sem_n3, sem_o0, sem_o1, sem_o2, sem_o3):
    wid = lax.axis_index("s") * 2 + lax.axis_index("c")
    base = wid * _P

    pltpu.sync_copy(nni.at[pl.ds(base * _K, _P * _K)], idx_v)
    pltpu.sync_copy(fli.at[pl.ds(base * _K, _P * _K)], fid_v)
    pltpu.sync_copy(c16.at[pl.ds(wid * _NG, _NG)], cnt_v)
    pltpu.sync_copy(swr, swl)
    lanes = lax.iota(jnp.int32, 16)
    sem_n = (sem_n0, sem_n1, sem_n2, sem_n3)
    sem_o = (sem_o0, sem_o1, sem_o2, sem_o3)

    def n_point(g, j, sl, sz):
      return pltpu.make_async_copy(
          inp.at[idx_v.at[pl.ds((g * _G + j) * _K, sz)]],
          nb.at[sl].at[pl.ds(j * _K, sz)], sem_n[sl])

    def n_each(g, sl, fn):
      cv = cnt_v[g]
      for j in range(_G):
        nr = (cv[j] + 7) & ~7
        for sz in (8, 16, 24, 32):

          @pl.when(nr == sz)
          def _(g=g, j=j, sl=sl, sz=sz):
            fn(n_point(g, j, sl, sz))

    def n_start(g, sl):
      n_each(g, sl, lambda c: c.start())

    def n_wait(g, sl):
      n_each(g, sl, lambda c: c.wait())

    def o_copy(g, sl):
      return pltpu.make_async_copy(
          ob.at[sl], out.at[pl.ds(base + g * _G, _G)], sem_o[sl])

    for g0 in range(4):
      n_start(g0, g0)

    def quad(h, carry):
      for sl in range(4):
        g = 4 * h + sl
        n_wait(g, sl)

        @pl.when(g >= 4)
        def _(g=g, sl=sl):
          o_copy(g - 4, sl).wait()

        cvec = cnt_v[g]
        for j in range(_G):
          cnt = cvec[j]
          p32 = (g * _G + j) * _K
          fv0 = fid_v[pl.ds(p32, 16)]
          fv1 = fid_v[pl.ds(p32 + 16, 16)]
          accs = tuple(jnp.zeros((16,), jnp.float32) for _ in range(8))

          def kbody(k, a, fv, koff, j=j, sl=sl):
            row = j * _K + koff + k
            fb = fv.at[jnp.full((16,), k, jnp.int32)].get(
                mode="promise_in_bounds")
            fbase = fb * _C + lanes
            return tuple(
                a[cb] + nb[sl, row, pl.ds(cb * 16, 16)]
                * plsc.load_gather(swl, [fbase + cb * 16])
                for cb in range(8))

          @pl.loop(0, jnp.minimum(cnt, 16), init_carry=accs)
          def accs(k, a, kb=kbody, fv0=fv0):
            return kb(k, a, fv0, 0)

          @pl.loop(0, jnp.maximum(cnt - 16, 0), init_carry=accs)
          def accs(k, a, kb=kbody, fv1=fv1):
            return kb(k, a, fv1, 16)

          for cb in range(8):
            ob[sl, j, pl.ds(cb * 16, 16)] = accs[cb]

        o_copy(g, sl).start()

        @pl.when(g + 4 < _NG)
        def _(g=g, sl=sl):
          n_start(g + 4, sl)
      return carry

    lax.fori_loop(0, _NG // 4, quad, 0)
    for g0 in range(4):
      o_copy(_NG - 4 + g0, g0).wait()

  return body(inputs, nnidx, filt, cnt16, sw)


def _tc_head(x, dw, b, gamma, beta):
  def body(x_ref, w_ref, b_ref, g_ref, bt_ref, o_ref):
    y = jnp.dot(x_ref[...], w_ref[...], preferred_element_type=jnp.float32)
    y = jnp.maximum(y + b_ref[...], 0.0)
    mean = jnp.mean(y, axis=0, keepdims=True)
    d = y - mean
    var = jnp.mean(d * d, axis=0, keepdims=True)
    o_ref[...] = d * lax.rsqrt(var + 1e-5) * g_ref[...] + bt_ref[...]

  return pl.pallas_call(
      body,
      out_shape=jax.ShapeDtypeStruct((_N, _OC), jnp.float32),
  )(x, dw, b, gamma, beta)


_POS2CH = [(p // 32) * 32 + 2 * (p % 16) + (p % 32) // 16 for p in range(_C)]


def kernel(inputs, nn_count, nn_index, filt_index, spatial_weights,
           depth_weights, biases, gamma, beta):
  pad = _NPAD - _N
  nni = jnp.pad(nn_index, ((0, pad), (0, 0))).reshape(-1)
  fli = jnp.pad(filt_index, ((0, pad), (0, 0))).reshape(-1)
  cnt = jnp.minimum(jnp.pad(nn_count, (0, pad)), _K)
  cnt16 = jnp.pad(cnt.reshape(-1, _G), ((0, 0), (0, 16 - _G)))
  conv = _sc_conv(inputs, nni, fli, cnt16, spatial_weights.reshape(-1))
  return _tc_head(conv[:_N], depth_weights, biases,
                  gamma.reshape(1, -1), beta.reshape(1, -1))


# speedup vs baseline: 1.4608x; 1.4608x over previous
import functools

import jax
import jax.numpy as jnp
from jax import lax
from jax.experimental import pallas as pl
from jax.experimental.pallas import tpu as pltpu
from jax.experimental.pallas import tpu_sc as plsc

_N, _K, _C, _OC, _KS = 10000, 32, 128, 128, 32
_NW = 32
_P = 320
_NPAD = _NW * _P
_G = 4
_NG = _P // _G
_R = _G * _K


def _sc_conv(inputs, nnidx, filt, cnt16, sw):
  mesh = plsc.VectorSubcoreMesh(core_axis_name="c", subcore_axis_name="s")

  @functools.partial(
      pl.kernel,
      mesh=mesh,
      out_type=jax.ShapeDtypeStruct((_NPAD, _C), jnp.float32),
      compiler_params=pltpu.CompilerParams(needs_layout_passes=False),
      scratch_types=[
          pltpu.VMEM((_P * _K,), jnp.int32),
          pltpu.VMEM((_P * _K,), jnp.int32),
          pltpu.VMEM((_NG, 16), jnp.int32),
          pltpu.VMEM((_KS * _C,), jnp.float32),
          pltpu.VMEM((2, _R, _C), jnp.float32),
          pltpu.VMEM((2, _G, _C), jnp.float32),
          pltpu.SemaphoreType.DMA,
          pltpu.SemaphoreType.DMA,
          pltpu.SemaphoreType.DMA,
          pltpu.SemaphoreType.DMA,
      ],
  )
  def body(inp, nni, fli, c16, swr, out, idx_v, fid_v, cnt_v, swl, nb, ob,
           sem_n0, sem_n1, sem_o0, sem_o1):
    wid = lax.axis_index("s") * 2 + lax.axis_index("c")
    base = wid * _P

    _TAIL = (_N - (_NW - 1) * _P) * _K

    @pl.when(wid < _NW - 1)
    def _():
      pltpu.sync_copy(nni.at[pl.ds(base * _K, _P * _K)], idx_v)
      pltpu.sync_copy(fli.at[pl.ds(base * _K, _P * _K)], fid_v)

    @pl.when(wid == _NW - 1)
    def _():
      pltpu.sync_copy(nni.at[pl.ds(base * _K, _TAIL)],
                      idx_v.at[pl.ds(0, _TAIL)])
      pltpu.sync_copy(fli.at[pl.ds(base * _K, _TAIL)],
                      fid_v.at[pl.ds(0, _TAIL)])
    pltpu.sync_copy(c16.at[pl.ds(wid * _NG, _NG)], cnt_v)
    pltpu.sync_copy(swr, swl)
    lanes = lax.iota(jnp.int32, 16)
    sem_n = (sem_n0, sem_n1)
    sem_o = (sem_o0, sem_o1)

    def n_point(g, j, sl, sz):
      return pltpu.make_async_copy(
          inp.at[idx_v.at[pl.ds((g * _G + j) * _K, sz)]],
          nb.at[sl].at[pl.ds(j * _K, sz)], sem_n[sl])

    def n_each(g, sl, fn):
      cv = cnt_v[g]
      for j in range(_G):
        nr = (cv[j] + 7) & ~7
        for sz in (8, 16, 24, 32):

          @pl.when(nr == sz)
          def _(g=g, j=j, sl=sl, sz=sz):
            fn(n_point(g, j, sl, sz))

    def n_start(g, sl):
      n_each(g, sl, lambda c: c.start())

    def n_wait(g, sl):
      n_each(g, sl, lambda c: c.wait())

    def o_copy(g, sl):
      return pltpu.make_async_copy(
          ob.at[sl], out.at[pl.ds(base + g * _G, _G)], sem_o[sl])

    n_start(0, 0)
    n_start(1, 1)

    def pair(h, carry):
      for sl in range(2):
        g = 2 * h + sl
        n_wait(g, sl)

        @pl.when(g >= 2)
        def _(g=g, sl=sl):
          o_copy(g - 2, sl).wait()

        cvec = cnt_v[g]
        for j in range(_G):
          cnt = cvec[j]
          p32 = (g * _G + j) * _K
          fv0 = fid_v[pl.ds(p32, 16)]
          fv1 = fid_v[pl.ds(p32 + 16, 16)]
          accs = tuple(jnp.zeros((16,), jnp.float32) for _ in range(8))

          def kbody(k, a, fv, koff, j=j, sl=sl):
            row = j * _K + koff + k
            fb = fv.at[jnp.full((16,), k, jnp.int32)].get(
                mode="promise_in_bounds")
            fbase = fb * _C + lanes
            return tuple(
                a[cb] + nb[sl, row, pl.ds(cb * 16, 16)]
                * plsc.load_gather(swl, [fbase + cb * 16])
                for cb in range(8))

          @pl.loop(0, jnp.minimum(cnt, 16), init_carry=accs)
          def accs(k, a, kb=kbody, fv0=fv0):
            return kb(k, a, fv0, 0)

          @pl.loop(0, jnp.maximum(cnt - 16, 0), init_carry=accs)
          def accs(k, a, kb=kbody, fv1=fv1):
            return kb(k, a, fv1, 16)

          for cb in range(8):
            ob[sl, j, pl.ds(cb * 16, 16)] = accs[cb]

        o_copy(g, sl).start()

        @pl.when(g + 2 < _NG)
        def _(g=g, sl=sl):
          n_start(g + 2, sl)
      return carry

    lax.fori_loop(0, _NG // 2, pair, 0)
    o_copy(_NG - 2, 0).wait()
    o_copy(_NG - 1, 1).wait()

  return body(inputs, nnidx, filt, cnt16, sw)


def _tc_head(x, dw, b, gamma, beta):
  def body(x_ref, w_ref, b_ref, g_ref, bt_ref, o_ref):
    y = jnp.dot(x_ref[pl.ds(0, _N), :], w_ref[...],
                preferred_element_type=jnp.float32)
    y = jnp.maximum(y + b_ref[...], 0.0)
    mean = jnp.mean(y, axis=0, keepdims=True)
    d = y - mean
    var = jnp.mean(d * d, axis=0, keepdims=True)
    o_ref[...] = d * lax.rsqrt(var + 1e-5) * g_ref[...] + bt_ref[...]

  return pl.pallas_call(
      body,
      out_shape=jax.ShapeDtypeStruct((_N, _OC), jnp.float32),
  )(x, dw, b, gamma, beta)


def kernel(inputs, nn_count, nn_index, filt_index, spatial_weights,
           depth_weights, biases, gamma, beta):
  pad = _NPAD - _N
  cnt = jnp.minimum(jnp.pad(nn_count, (0, pad)), _K)
  cnt16 = jnp.pad(cnt.reshape(-1, _G), ((0, 0), (0, 16 - _G)))
  conv = _sc_conv(inputs, nn_index.reshape(-1), filt_index.reshape(-1),
                  cnt16, spatial_weights.reshape(-1))
  return _tc_head(conv, depth_weights, biases,
                  gamma.reshape(1, -1), beta.reshape(1, -1))
